# Initial kernel scaffold; baseline (speedup 1.0000x reference)
import jax
import jax.numpy as jnp
from jax.experimental import pallas as pl


def kernel(x, edge_index, batch, W_lc1, b_lc1, W_lc2, b_lc2, Wc, bc, gc, betac, rmc, rvc, Wo1, bo1, Wo2, bo2, Wo3, bo3):
    raise NotImplementedError("write your pallas kernel here")


# speedup vs baseline: 1.8666x; 1.8666x over previous
import functools

import jax
import jax.numpy as jnp
from jax import lax
from jax.experimental import pallas as pl
from jax.experimental.pallas import tpu as pltpu
from jax.experimental.pallas import tpu_sc as plsc

N = 50000
E = 800000
H = 64

HALF = 25088
NPAD = 2 * HALF
RPT = HALF // 16
TRASH = HALF
ACCR = HALF + 8
C = 128
EP = E + 2048
FILLDST = 1 << 30

f32 = jnp.float32
i32 = jnp.int32


@functools.lru_cache(maxsize=None)
def _make_edge_pass(with_deg):
  mesh = plsc.VectorSubcoreMesh(
      core_axis_name="c", subcore_axis_name="s", num_cores=2, num_subcores=16)

  out_type = [jax.ShapeDtypeStruct((NPAD, H), f32)]
  scratch = [
      pltpu.VMEM((16,), i32),
      pltpu.VMEM((H,), f32),
      pltpu.VMEM((H,), f32),
      pltpu.VMEM((C,), i32),
      pltpu.VMEM((C,), i32),
      pltpu.VMEM((C,), i32),
      pltpu.VMEM((C,), i32),
      pltpu.VMEM((C, H), f32),
      pltpu.VMEM((C, H), f32),
      pltpu.VMEM_SHARED((ACCR, H), f32),
      pltpu.SemaphoreType.DMA,
      pltpu.SemaphoreType.DMA,
  ]
  if with_deg:
    out_type.append(jax.ShapeDtypeStruct((NPAD, H), f32))

  def body(a_h, b_h, sdst_h, ssrc_h, qinfo_h, sc_h, sh_h, *refs):
    if with_deg:
      (acc_out, deg_out, qv, scv, shv, dstv, srcv, gidxv, sidxv,
       arow, brow, acc, sem_a, sem_b) = refs
    else:
      (acc_out, qv, scv, shv, dstv, srcv, gidxv, sidxv,
       arow, brow, acc, sem_a, sem_b) = refs
      deg_out = None

    c = lax.axis_index("c")
    t = lax.axis_index("s")

    pltpu.sync_copy(qinfo_h, qv)
    iot16 = lax.iota(i32, 16)
    vq = qv[pl.ds(0, 16)]
    k0 = jnp.sum(jnp.where(iot16 == 0, vq, 0))
    q1 = jnp.sum(jnp.where(iot16 == 1, vq, 0))
    k1 = jnp.sum(jnp.where(iot16 == 2, vq, 0))

    pltpu.sync_copy(sc_h, scv)
    pltpu.sync_copy(sh_h, shv)
    scale = [scv[pl.ds(f * 16, 16)] for f in range(4)]
    shift = [shv[pl.ds(f * 16, 16)] for f in range(4)]

    def fill_arow(val):
      def frow(r, carry):
        for f in range(4):
          arow[r, pl.ds(f * 16, 16)] = jnp.full((16,), val, f32)
        return carry
      lax.fori_loop(0, C, frow, 0)

    zbase = t * RPT

    def zero_acc():
      for i in range(12):
        pltpu.sync_copy(arow, acc.at[pl.ds(zbase + i * C, C)])
      pltpu.sync_copy(arow.at[pl.ds(0, 32)],
                      acc.at[pl.ds(zbase + 12 * C, 32)])

    start = c * q1
    nn = (1 - c) * k0 + c * k1
    pt = ((nn + 15) // 16 + 7) // 8 * 8
    base = start + t * pt
    limit = jnp.minimum(start + nn, base + pt)
    nchunks = (pt + C - 1) // C
    chalf = c * HALF
    iot = lax.iota(i32, 16)

    if with_deg:
      fill_arow(0.0)
      zero_acc()
      plsc.subcore_barrier()
      fill_arow(1.0)

      def dchunk(k, carry):
        p0 = pl.multiple_of(base + k * C, 8)
        pltpu.async_copy(sdst_h.at[pl.ds(p0, C)], dstv, sem_a).wait()
        for j in range(C // 16):
          d = dstv[pl.ds(j * 16, 16)]
          loc = d - chalf
          pos = (p0 + j * 16) + iot
          valid = (loc >= 0) & (loc < HALF) & (pos < limit)
          sidxv[pl.ds(j * 16, 16)] = jnp.where(valid, loc, TRASH)
        pltpu.sync_copy(arow, acc.at[sidxv], add=True)
        return carry
      lax.fori_loop(0, nchunks, dchunk, 0)

      plsc.subcore_barrier()
      pltpu.sync_copy(acc.at[pl.ds(t * RPT, RPT)],
                      deg_out.at[pl.ds(chalf + t * RPT, RPT)])

    fill_arow(0.0)
    zero_acc()
    plsc.subcore_barrier()

    def chunk(k, carry):
      p0 = pl.multiple_of(base + k * C, 8)
      cp1 = pltpu.async_copy(sdst_h.at[pl.ds(p0, C)], dstv, sem_a)
      cp2 = pltpu.async_copy(ssrc_h.at[pl.ds(p0, C)], srcv, sem_b)
      cp1.wait()
      cp2.wait()
      for j in range(C // 16):
        d = dstv[pl.ds(j * 16, 16)]
        loc = d - chalf
        pos = (p0 + j * 16) + iot
        valid = (loc >= 0) & (loc < HALF) & (pos < limit)
        sidxv[pl.ds(j * 16, 16)] = jnp.where(valid, loc, TRASH)
        gidxv[pl.ds(j * 16, 16)] = jnp.where(valid, d, 0)
      cpa = pltpu.async_copy(a_h.at[gidxv], arow, sem_a)
      cpb = pltpu.async_copy(b_h.at[srcv], brow, sem_b)
      cpa.wait()
      cpb.wait()

      def mrow(r, cc):
        for f in range(4):
          sl = pl.ds(f * 16, 16)
          y = arow[r, sl] + brow[r, sl]
          m = jnp.where(y > 0.0, y, jnp.exp(y) - 1.0)
          arow[r, sl] = m * scale[f] + shift[f]
        return cc
      lax.fori_loop(0, C, mrow, 0)

      pltpu.sync_copy(arow, acc.at[sidxv], add=True)
      return carry
    lax.fori_loop(0, nchunks, chunk, 0)

    plsc.subcore_barrier()

    ob = chalf + t * RPT
    pltpu.sync_copy(acc.at[pl.ds(t * RPT, RPT)], acc_out.at[pl.ds(ob, RPT)])

  return pl.kernel(body, out_type=out_type, mesh=mesh,
                   scratch_types=scratch, name="edge_pass",
                   compiler_params=pltpu.CompilerParams(
                       use_tc_tiling_on_sc=False,
                       needs_layout_passes=False))


_R = 3136
_GRID = NPAD // _R


def _elu(x):
  return jnp.where(x > 0.0, x, jnp.exp(x) - 1.0)


def _rows_spec(w):
  return pl.BlockSpec((_R, w), lambda i: (i, 0))


def _full_spec(r, w):
  return pl.BlockSpec((r, w), lambda i: (0, 0))


def _tc0_body(x_ref, w1_ref, b1_ref, w2_ref, b2_ref, wd_ref, wb_ref, bc_ref,
              h_ref, a_ref, b_ref):
  x = x_ref[...]
  h = _elu(jnp.dot(x, w1_ref[...], preferred_element_type=f32) + b1_ref[...])
  h = _elu(jnp.dot(h, w2_ref[...], preferred_element_type=f32) + b2_ref[...])
  h_ref[...] = h
  a_ref[...] = jnp.dot(h, wd_ref[...], preferred_element_type=f32) + bc_ref[...]
  b_ref[...] = jnp.dot(h, wb_ref[...], preferred_element_type=f32)


def _tc0(xpad, w1, b1, w2, b2, wd, wb, bc_):
  return pl.pallas_call(
      _tc0_body,
      grid=(_GRID,),
      in_specs=[_rows_spec(15), _full_spec(15, H), _full_spec(1, H),
                _full_spec(H, H), _full_spec(1, H), _full_spec(H, H),
                _full_spec(H, H), _full_spec(1, H)],
      out_specs=[_rows_spec(H), _rows_spec(H), _rows_spec(H)],
      out_shape=[jax.ShapeDtypeStruct((NPAD, H), f32)] * 3,
  )(xpad, w1, b1, w2, b2, wd, wb, bc_)


def _tcmid_body(acc_ref, deg_ref, h_ref, wd_ref, wb_ref, bc_ref,
                hn_ref, a_ref, b_ref):
  dinv = 1.0 / jnp.maximum(deg_ref[...][:, 0:1], 1.0)
  hn = acc_ref[...] * dinv + h_ref[...]
  hn_ref[...] = hn
  a_ref[...] = jnp.dot(hn, wd_ref[...], preferred_element_type=f32) + bc_ref[...]
  b_ref[...] = jnp.dot(hn, wb_ref[...], preferred_element_type=f32)


def _tcmid(acc, deg, h, wd, wb, bc_):
  return pl.pallas_call(
      _tcmid_body,
      grid=(_GRID,),
      in_specs=[_rows_spec(H), _rows_spec(H), _rows_spec(H),
                _full_spec(H, H), _full_spec(H, H), _full_spec(1, H)],
      out_specs=[_rows_spec(H), _rows_spec(H), _rows_spec(H)],
      out_shape=[jax.ShapeDtypeStruct((NPAD, H), f32)] * 3,
  )(acc, deg, h, wd, wb, bc_)


def _tc3_body(acc_ref, deg_ref, h_ref, wo1_ref, bo1_ref, wo2_ref, bo2_ref,
              wo3_ref, bo3_ref, o_ref):
  dinv = 1.0 / jnp.maximum(deg_ref[...][:, 0:1], 1.0)
  hn = acc_ref[...] * dinv + h_ref[...]
  o = _elu(jnp.dot(hn, wo1_ref[...], preferred_element_type=f32) + bo1_ref[...])
  o = _elu(jnp.dot(o, wo2_ref[...], preferred_element_type=f32) + bo2_ref[...])
  o_ref[...] = jnp.dot(o, wo3_ref[...], preferred_element_type=f32) + bo3_ref[...]


def _tc3(acc, deg, h, wo1, bo1, wo2, bo2, wo3, bo3):
  return pl.pallas_call(
      _tc3_body,
      grid=(_GRID,),
      in_specs=[_rows_spec(H), _rows_spec(H), _rows_spec(H),
                _full_spec(H, 32), _full_spec(1, 32), _full_spec(32, 16),
                _full_spec(1, 16), _full_spec(16, 8), _full_spec(1, 8)],
      out_specs=[_rows_spec(8)],
      out_shape=[jax.ShapeDtypeStruct((NPAD, 8), f32)],
  )(acc, deg, h, wo1, bo1, wo2, bo2, wo3, bo3)[0]


def kernel(x, edge_index, batch, W_lc1, b_lc1, W_lc2, b_lc2, Wc, bc, gc,
           betac, rmc, rvc, Wo1, bo1, Wo2, bo2, Wo3, bo3):
  src = edge_index[0]
  dst = edge_index[1]

  m0 = dst < HALF
  cs = jnp.cumsum(m0.astype(i32))
  k0 = cs[-1]
  q1 = (k0 + 127) // 128 * 128
  ar = jnp.arange(E, dtype=i32)
  pos = jnp.where(m0, cs - 1, q1 + (ar - cs))
  sdst = jnp.full((EP,), FILLDST, i32).at[pos].set(dst)
  ssrc = jnp.zeros((EP,), i32).at[pos].set(src)
  qinfo = (jnp.zeros((16,), i32)
           .at[0].set(k0).at[1].set(q1).at[2].set(E - k0))

  xpad = jnp.pad(x, ((0, NPAD - N), (0, 0)))
  scale = gc / jnp.sqrt(rvc + 1e-5)
  shift = betac - rmc * scale
  wd = Wc[:, :H, :] - Wc[:, H:, :]
  wb = Wc[:, H:, :]

  h, a, bm = _tc0(xpad, W_lc1, b_lc1.reshape(1, H), W_lc2,
                  b_lc2.reshape(1, H), wd[0], wb[0], bc[0].reshape(1, H))
  acc, deg = _make_edge_pass(True)(a, bm, sdst, ssrc, qinfo,
                                   scale[0], shift[0])
  for i in (1, 2):
    h, a, bm = _tcmid(acc, deg, h, wd[i], wb[i], bc[i].reshape(1, H))
    acc = _make_edge_pass(False)(a, bm, sdst, ssrc, qinfo,
                                 scale[i], shift[i])[0]

  o = _tc3(acc, deg, h, Wo1, bo1.reshape(1, 32), Wo2, bo2.reshape(1, 16),
           Wo3, bo3.reshape(1, 8))
  return o[:N], batch
